# baseline (speedup 1.0000x reference)
import functools

import jax
import jax.numpy as jnp
from jax.experimental import pallas as pl
from jax.experimental.pallas import tpu as pltpu

_TILE_N = 2048


def _fused_kernel(xa_ref, xb_ref, embt_ref, scale_ref, bias_ref, out_ref, idx_ref):
    @pl.when(pl.program_id(0) == 0)
    def _():
        idx_ref[...] = jax.lax.broadcasted_iota(jnp.int32, idx_ref.shape, 0)
    embt = embt_ref[...]
    scale = scale_ref[...].reshape(-1, 1)
    bias = bias_ref[...].reshape(-1, 1)
    half = _TILE_N // 2
    for j, x_ref in enumerate((xa_ref, xb_ref)):
        xt = x_ref[...]
        out = jax.lax.dot_general(
            embt, xt, (((1,), (0,)), ((), ())), preferred_element_type=jnp.float32
        )
        out = 0.5 * out * (1.0 + jax.lax.erf(out * (2.0 ** -0.5)))
        mu = jnp.mean(out, axis=0, keepdims=True)
        var = jnp.mean((out - mu) ** 2, axis=0, keepdims=True)
        out = (out - mu) / jnp.sqrt(var + 1e-5) * scale + bias
        out_ref[:, pl.ds(j * half, half)] = out


@functools.partial(jax.jit, static_argnames=())
def kernel(x, emb, ln_scale, ln_bias):
    batch, num_genes = x.shape
    num_hidden = emb.shape[1]
    xt = x.T
    embt = emb.T
    grid = (batch // _TILE_N,)
    out_t, gene_idx = pl.pallas_call(
        _fused_kernel,
        grid=grid,
        in_specs=[
            pl.BlockSpec((num_genes, _TILE_N // 2), lambda i: (0, 2 * i)),
            pl.BlockSpec((num_genes, _TILE_N // 2), lambda i: (0, 2 * i + 1)),
            pl.BlockSpec((num_hidden, num_genes), lambda i: (0, 0)),
            pl.BlockSpec((1, num_hidden), lambda i: (0, 0)),
            pl.BlockSpec((1, num_hidden), lambda i: (0, 0)),
        ],
        out_specs=[
            pl.BlockSpec((num_hidden, _TILE_N), lambda i: (0, i)),
            pl.BlockSpec((num_genes,), lambda i: (0,)),
        ],
        out_shape=[
            jax.ShapeDtypeStruct((num_hidden, batch), jnp.float32),
            jax.ShapeDtypeStruct((num_genes,), jnp.int32),
        ],
        compiler_params=pltpu.CompilerParams(
            dimension_semantics=("parallel",),
        ),
    )(xt, xt, embt, ln_scale.reshape(1, num_hidden), ln_bias.reshape(1, num_hidden))
    return (out_t.T, gene_idx)


# speedup vs baseline: 1.0030x; 1.0030x over previous
import functools

import jax
import jax.numpy as jnp
from jax.experimental import pallas as pl
from jax.experimental.pallas import tpu as pltpu

_TILE_N = 2048


def _fused_kernel(xt_ref, embt_ref, scale_ref, bias_ref, out_ref, idx_ref):
    @pl.when(pl.program_id(0) == 0)
    def _():
        idx_ref[...] = jax.lax.broadcasted_iota(jnp.int32, idx_ref.shape, 0)
    xt = xt_ref[...]
    embt = embt_ref[...]
    out = jax.lax.dot_general(
        embt, xt, (((1,), (0,)), ((), ())), preferred_element_type=jnp.float32
    )
    out = 0.5 * out * (1.0 + jax.lax.erf(out * (2.0 ** -0.5)))
    mu = jnp.mean(out, axis=0, keepdims=True)
    var = jnp.mean((out - mu) ** 2, axis=0, keepdims=True)
    scale = scale_ref[...].reshape(-1, 1)
    bias = bias_ref[...].reshape(-1, 1)
    out = (out - mu) / jnp.sqrt(var + 1e-5) * scale + bias
    out_ref[...] = out


@functools.partial(jax.jit, static_argnames=())
def kernel(x, emb, ln_scale, ln_bias):
    batch, num_genes = x.shape
    num_hidden = emb.shape[1]
    xt = x.T
    embt = emb.T
    grid = (batch // _TILE_N,)
    out_t, gene_idx = pl.pallas_call(
        _fused_kernel,
        grid=grid,
        in_specs=[
            pl.BlockSpec((num_genes, _TILE_N), lambda i: (0, i)),
            pl.BlockSpec((num_hidden, num_genes), lambda i: (0, 0)),
            pl.BlockSpec((1, num_hidden), lambda i: (0, 0)),
            pl.BlockSpec((1, num_hidden), lambda i: (0, 0)),
        ],
        out_specs=[
            pl.BlockSpec((num_hidden, _TILE_N), lambda i: (0, i)),
            pl.BlockSpec((num_genes,), lambda i: (0,)),
        ],
        out_shape=[
            jax.ShapeDtypeStruct((num_hidden, batch), jnp.float32),
            jax.ShapeDtypeStruct((num_genes,), jnp.int32),
        ],
        compiler_params=pltpu.CompilerParams(
            dimension_semantics=("parallel",),
        ),
    )(xt, embt, ln_scale.reshape(1, num_hidden), ln_bias.reshape(1, num_hidden))
    return (out_t.T, gene_idx)
